# baseline (speedup 1.0000x reference)
import functools

import jax
import jax.numpy as jnp
from jax.experimental import pallas as pl
from jax.experimental.pallas import tpu as pltpu
from jax.experimental.pallas import tpu_sc as plsc

BN_EPS = 1e-5

NC = 2
NS = 16
LANES = 16
WIN = 128
NBUF = 2
SDEPTH = 4


def _relu_tc(x):
    def body(x_ref, o_ref):
        o_ref[...] = jnp.maximum(x_ref[...], 0.0)

    return pl.pallas_call(
        body, out_shape=jax.ShapeDtypeStruct(x.shape, x.dtype))(x)


def _sc_segment_sum(relu_x, idx_flat, windows, n_pad):
    n, d = relu_x.shape
    rows_per_subcore = n_pad // NS
    mesh = plsc.VectorSubcoreMesh(core_axis_name="c", subcore_axis_name="s")

    @functools.partial(
        pl.kernel,
        out_type=jax.ShapeDtypeStruct((NC, n_pad, d), jnp.float32),
        mesh=mesh,
        scratch_types=[
            pltpu.VMEM((SDEPTH, 2, WIN), jnp.int32),
            pltpu.VMEM((NBUF, WIN, d), jnp.float32),
            pltpu.VMEM_SHARED((n_pad, d), jnp.float32),
        ] + [pltpu.SemaphoreType.DMA] * (NBUF + NBUF + SDEPTH),
    )
    def k(relu_x_hbm, idx_hbm, out_hbm, stage, bufs, acc, *sems):
        c = jax.lax.axis_index("c")
        s = jax.lax.axis_index("s")
        wid = c * NS + s
        gsem = sems[:NBUF]
        ssem = sems[NBUF:2 * NBUF]
        isem = sems[2 * NBUF:]

        def start_idx(w, q):
            base = (wid * windows + w) * (2 * WIN)
            pltpu.async_copy(idx_hbm.at[pl.ds(base, WIN)],
                             stage.at[q, 0], isem[q])
            pltpu.async_copy(idx_hbm.at[pl.ds(base + WIN, WIN)],
                             stage.at[q, 1], isem[q])

        def wait_idx(q):
            pltpu.make_async_copy(idx_hbm.at[pl.ds(0, WIN)],
                                  stage.at[q, 0], isem[q]).wait()
            pltpu.make_async_copy(idx_hbm.at[pl.ds(0, WIN)],
                                  stage.at[q, 1], isem[q]).wait()

        def start_gather(p, q):
            return pltpu.async_copy(
                relu_x_hbm.at[stage.at[q, 0]], bufs.at[p], gsem[p])

        def wait_gather(p, q):
            pltpu.make_async_copy(
                relu_x_hbm.at[stage.at[q, 0]], bufs.at[p], gsem[p]).wait()

        def start_scatter(p, q):
            return pltpu.async_copy(
                bufs.at[p], acc.at[stage.at[q, 1]], ssem[p], add=True)

        def wait_scatter(p, q):
            pltpu.make_async_copy(
                bufs.at[p], acc.at[stage.at[q, 1]], ssem[p]).wait()

        @pl.loop(0, WIN)
        def _(r):
            @pl.loop(0, d, step=LANES)
            def _(col):
                bufs.at[0, r, pl.ds(col, LANES)][...] = jnp.zeros(
                    (LANES,), jnp.float32)

        row0 = s * rows_per_subcore

        @pl.loop(0, rows_per_subcore, step=WIN)
        def _(r):
            pltpu.sync_copy(bufs.at[0], acc.at[pl.ds(row0 + r, WIN)])

        plsc.subcore_barrier()

        start_idx(0, 0)
        start_idx(1, 1)

        @pl.loop(0, windows, step=2 * NBUF)
        def _(base):
            for j in range(2 * NBUF):
                w = base + j
                p, q = j % NBUF, j % SDEPTH

                @pl.when(w >= 2)
                def _():
                    wait_scatter(p, (j + 2) % SDEPTH)

                @pl.when(w + 2 < windows)
                def _():
                    start_idx(w + 2, (j + 2) % SDEPTH)

                wait_idx(q)
                start_gather(p, q)
                wait_gather(p, q)
                start_scatter(p, q)

        wait_scatter(0, (windows - 2) % SDEPTH)
        wait_scatter(1, (windows - 1) % SDEPTH)

        plsc.subcore_barrier()
        pltpu.sync_copy(acc.at[pl.ds(row0, rows_per_subcore)],
                        out_hbm.at[c].at[pl.ds(row0, rows_per_subcore)])

    return k(relu_x, idx_flat)


def _mlp_tc(x, parts, W1, b1, gamma, beta, W2, b2, eps_param):
    n, d = x.shape
    h1_dim = W1.shape[1]

    def body(x_ref, p_ref, w1_ref, b1_ref, g_ref, be_ref, w2_ref, b2_ref,
             eps_ref, o_ref):
        h = (x_ref[...] * (1.0 + eps_ref[0, 0])
             + p_ref[0, :n, :] + p_ref[1, :n, :])
        h1 = jnp.dot(h, w1_ref[...],
                     preferred_element_type=jnp.float32) + b1_ref[...]
        mean = jnp.mean(h1, axis=0, keepdims=True)
        var = jnp.mean((h1 - mean) ** 2, axis=0, keepdims=True)
        h1 = (h1 - mean) / jnp.sqrt(var + BN_EPS) * g_ref[...] + be_ref[...]
        h1 = jnp.maximum(h1, 0.0)
        o_ref[...] = jnp.dot(h1, w2_ref[...],
                             preferred_element_type=jnp.float32) + b2_ref[...]

    return pl.pallas_call(
        body,
        out_shape=jax.ShapeDtypeStruct((n, W2.shape[1]), jnp.float32),
    )(x, parts, W1, b1.reshape(1, h1_dim), gamma.reshape(1, h1_dim),
      beta.reshape(1, h1_dim), W2, b2.reshape(1, W2.shape[1]),
      eps_param.reshape(1, 1))


def kernel(x, edge_index, W1, b1, gamma, beta, W2, b2, eps_param):
    n = x.shape[0]
    e = edge_index.shape[1]
    nw = NC * NS
    n_pad = -(-n // (NS * 128)) * (NS * 128)
    per_w = e // nw
    per_w_pad = -(-per_w // (4 * WIN)) * (4 * WIN)
    npad_e = per_w_pad - per_w
    windows = per_w_pad // WIN

    relu_x = _relu_tc(x)

    src_w = edge_index[0].reshape(nw, per_w)
    dst_w = edge_index[1].reshape(nw, per_w)
    if npad_e:
        lane = jnp.arange(nw, dtype=jnp.int32)[:, None]
        j = jnp.arange(npad_e, dtype=jnp.int32)[None, :]
        pad_src = (lane * 37 + j * 101) % n
        pad_dst = n + (lane * 7 + j) % (n_pad - n)
        src_w = jnp.concatenate([src_w, pad_src], axis=1)
        dst_w = jnp.concatenate([dst_w, pad_dst], axis=1)
    idx_flat = jnp.stack(
        [src_w.reshape(nw, windows, WIN), dst_w.reshape(nw, windows, WIN)],
        axis=2).reshape(-1)

    parts = _sc_segment_sum(relu_x, idx_flat, windows, n_pad)
    return _mlp_tc(x, parts, W1, b1, gamma, beta, W2, b2, eps_param)


# speedup vs baseline: 1.2337x; 1.2337x over previous
import functools

import jax
import jax.numpy as jnp
from jax.experimental import pallas as pl
from jax.experimental.pallas import tpu as pltpu
from jax.experimental.pallas import tpu_sc as plsc

BN_EPS = 1e-5

NC = 2
NS = 16
LANES = 16
WIN = 80
NBUF = 3


def _relu_tc(x):
    def body(x_ref, o_ref):
        o_ref[...] = jnp.maximum(x_ref[...], 0.0)

    return pl.pallas_call(
        body, out_shape=jax.ShapeDtypeStruct(x.shape, x.dtype))(x)


def _sc_segment_sum(relu_x, packed3d, n_pad):
    n, d = relu_x.shape
    windows = packed3d.shape[1]
    rows_per_subcore = n_pad // NS
    mesh = plsc.VectorSubcoreMesh(core_axis_name="c", subcore_axis_name="s")

    @functools.partial(
        pl.kernel,
        out_type=jax.ShapeDtypeStruct((NC, n_pad, d), jnp.float32),
        mesh=mesh,
        scratch_types=[
            pltpu.VMEM((windows, WIN), jnp.int32),
            pltpu.VMEM((NBUF, 2, WIN), jnp.int32),
            pltpu.VMEM((NBUF, WIN, d), jnp.float32),
            pltpu.VMEM_SHARED((n_pad, d), jnp.float32),
        ] + [pltpu.SemaphoreType.DMA] * (2 * NBUF),
    )
    def k(relu_x_hbm, pk_hbm, out_hbm, pk, stage, bufs, acc, *sems):
        c = jax.lax.axis_index("c")
        s = jax.lax.axis_index("s")
        wid = c * NS + s
        gsem = sems[:NBUF]
        ssem = sems[NBUF:]

        def unpack(w, p):
            @pl.loop(0, WIN, step=LANES)
            def _(j):
                v = pk.at[w, pl.ds(j, LANES)][...]
                stage.at[p, 0, pl.ds(j, LANES)][...] = (
                    jax.lax.shift_right_logical(v, 15))
                stage.at[p, 1, pl.ds(j, LANES)][...] = (
                    jax.lax.bitwise_and(v, 32767))

        def start_gather(p):
            return pltpu.async_copy(
                relu_x_hbm.at[stage.at[p, 0]], bufs.at[p], gsem[p])

        def wait_gather(p):
            pltpu.make_async_copy(
                relu_x_hbm.at[stage.at[p, 0]], bufs.at[p], gsem[p]).wait()

        def start_scatter(p):
            return pltpu.async_copy(
                bufs.at[p], acc.at[stage.at[p, 1]], ssem[p], add=True)

        def wait_scatter(p):
            pltpu.make_async_copy(
                bufs.at[p], acc.at[stage.at[p, 1]], ssem[p]).wait()

        @pl.loop(0, WIN)
        def _(r):
            @pl.loop(0, d, step=LANES)
            def _(col):
                bufs.at[0, r, pl.ds(col, LANES)][...] = jnp.zeros(
                    (LANES,), jnp.float32)

        row0 = s * rows_per_subcore

        @pl.loop(0, rows_per_subcore, step=WIN)
        def _(r):
            pltpu.sync_copy(bufs.at[0], acc.at[pl.ds(row0 + r, WIN)])

        pltpu.sync_copy(pk_hbm.at[wid], pk)
        plsc.subcore_barrier()

        unpack(0, 0)
        start_gather(0)

        @pl.loop(0, windows, step=NBUF)
        def _(base):
            for j in range(NBUF):
                w = base + j
                p = j
                p_next = (j + 1) % NBUF

                @pl.when(w >= 2)
                def _():
                    wait_scatter(p_next)

                @pl.when(w + 1 < windows)
                def _():
                    unpack(w + 1, p_next)
                    start_gather(p_next)

                wait_gather(p)
                start_scatter(p)

        wait_scatter((windows - 2) % NBUF)
        wait_scatter((windows - 1) % NBUF)

        plsc.subcore_barrier()
        pltpu.sync_copy(acc.at[pl.ds(row0, rows_per_subcore)],
                        out_hbm.at[c].at[pl.ds(row0, rows_per_subcore)])

    return k(relu_x, packed3d)


def _mlp_tc(x, parts, W1, b1, gamma, beta, W2, b2, eps_param):
    n, d = x.shape
    h1_dim = W1.shape[1]

    def body(x_ref, p_ref, w1_ref, b1_ref, g_ref, be_ref, w2_ref, b2_ref,
             eps_ref, o_ref):
        h = (x_ref[...] * (1.0 + eps_ref[0, 0])
             + p_ref[0, :n, :] + p_ref[1, :n, :])
        h1 = jnp.dot(h, w1_ref[...],
                     preferred_element_type=jnp.float32) + b1_ref[...]
        mean = jnp.mean(h1, axis=0, keepdims=True)
        var = jnp.mean((h1 - mean) ** 2, axis=0, keepdims=True)
        h1 = (h1 - mean) / jnp.sqrt(var + BN_EPS) * g_ref[...] + be_ref[...]
        h1 = jnp.maximum(h1, 0.0)
        o_ref[...] = jnp.dot(h1, w2_ref[...],
                             preferred_element_type=jnp.float32) + b2_ref[...]

    return pl.pallas_call(
        body,
        out_shape=jax.ShapeDtypeStruct((n, W2.shape[1]), jnp.float32),
    )(x, parts, W1, b1.reshape(1, h1_dim), gamma.reshape(1, h1_dim),
      beta.reshape(1, h1_dim), W2, b2.reshape(1, W2.shape[1]),
      eps_param.reshape(1, 1))


def kernel(x, edge_index, W1, b1, gamma, beta, W2, b2, eps_param):
    n = x.shape[0]
    e = edge_index.shape[1]
    nw = NC * NS
    n_pad = -(-n // (NS * 128)) * (NS * 128)
    per_w = e // nw
    per_w_pad = -(-per_w // (NBUF * WIN)) * (NBUF * WIN)
    npad_e = per_w_pad - per_w
    windows = per_w_pad // WIN

    relu_x = _relu_tc(x)

    src_w = edge_index[0].reshape(nw, per_w)
    dst_w = edge_index[1].reshape(nw, per_w)
    if npad_e:
        lane = jnp.arange(nw, dtype=jnp.int32)[:, None]
        j = jnp.arange(npad_e, dtype=jnp.int32)[None, :]
        pad_src = (lane * 37 + j * 101) % n
        pad_dst = n + (lane * 7 + j) % (n_pad - n)
        src_w = jnp.concatenate([src_w, pad_src], axis=1)
        dst_w = jnp.concatenate([dst_w, pad_dst], axis=1)
    packed3d = ((src_w << 15) | dst_w).reshape(nw, windows, WIN)

    parts = _sc_segment_sum(relu_x, packed3d, n_pad)
    return _mlp_tc(x, parts, W1, b1, gamma, beta, W2, b2, eps_param)
